# baseline (speedup 1.0000x reference)
import functools
import math

import jax
import jax.numpy as jnp
from jax import lax
from jax.experimental import pallas as pl
from jax.experimental.pallas import tpu as pltpu
from jax.experimental.pallas import tpu_sc as plsc

NUM_ROWS = 1000000
DIM = 64
BATCH = 4096
SEQ = 50

_info = plsc.get_sparse_core_info()
NC, NS = _info.num_cores, _info.num_subcores
NW = NC * NS

B = BATCH * SEQ
B_PER_W = B // NW
CHUNK = 128
CHUNKS = B_PER_W // CHUNK

_mesh = plsc.VectorSubcoreMesh(core_axis_name="c", subcore_axis_name="s")


@functools.partial(
    pl.kernel,
    out_type=jax.ShapeDtypeStruct((B, DIM), jnp.float32),
    mesh=_mesh,
    scratch_types=[
        pltpu.VMEM((CHUNKS, CHUNK), jnp.int32),
        pltpu.VMEM((CHUNK, DIM), jnp.float32),
        pltpu.SemaphoreType.DMA,
    ],
    compiler_params=pltpu.CompilerParams(use_tc_tiling_on_sc=False),
)
def _gather_kernel(idx_hbm, table_hbm, out_hbm, idx_v, buf, sem):
    wid = lax.axis_index("s") * NC + lax.axis_index("c")
    pltpu.sync_copy(idx_hbm.at[wid], idx_v)
    base = wid * B_PER_W

    @pl.loop(0, CHUNKS)
    def _chunk(j):
        pltpu.async_copy(table_hbm.at[idx_v.at[j]], buf, sem).wait()
        pltpu.sync_copy(buf, out_hbm.at[pl.ds(base + j * CHUNK, CHUNK)])


def kernel(input, weight):
    idx = input.reshape(NW, CHUNKS, CHUNK).astype(jnp.int32)
    out = _gather_kernel(idx, weight)
    return out.reshape(BATCH, SEQ, DIM)


# speedup vs baseline: 1.0441x; 1.0441x over previous
import functools
import math

import jax
import jax.numpy as jnp
from jax import lax
from jax.experimental import pallas as pl
from jax.experimental.pallas import tpu as pltpu
from jax.experimental.pallas import tpu_sc as plsc

NUM_ROWS = 1000000
DIM = 64
BATCH = 4096
SEQ = 50

_info = plsc.get_sparse_core_info()
NC, NS = _info.num_cores, _info.num_subcores
NW = NC * NS

B = BATCH * SEQ
B_PER_W = B // NW
CHUNK = 128
CHUNKS = B_PER_W // CHUNK

_mesh = plsc.VectorSubcoreMesh(core_axis_name="c", subcore_axis_name="s")


NBUF = 5
KLEAD = 2


@functools.partial(
    pl.kernel,
    out_type=jax.ShapeDtypeStruct((B, DIM), jnp.float32),
    mesh=_mesh,
    scratch_types=[
        pltpu.VMEM((CHUNKS, CHUNK), jnp.int32),
        pltpu.VMEM((NBUF, CHUNK, DIM), jnp.float32),
        pltpu.SemaphoreType.DMA((NBUF,)),
        pltpu.SemaphoreType.DMA((NBUF,)),
    ],
    compiler_params=pltpu.CompilerParams(use_tc_tiling_on_sc=False),
)
def _gather_kernel(idx_hbm, table_hbm, out_hbm, idx_v, bufs, gsem, wsem):
    wid = lax.axis_index("s") * NC + lax.axis_index("c")
    pltpu.sync_copy(idx_hbm.at[wid], idx_v)
    base = wid * B_PER_W

    for jj in range(KLEAD):
        pltpu.async_copy(
            table_hbm.at[idx_v.at[jj]], bufs.at[jj % NBUF], gsem.at[jj % NBUF]
        )

    @pl.loop(0, CHUNKS, step=NBUF)
    def _group(g):
        for b in range(NBUF):
            j = g + b
            gs = (b + KLEAD) % NBUF

            @pl.when(j + KLEAD < CHUNKS)
            def _():
                @pl.when(j + KLEAD >= NBUF)
                def _():
                    pltpu.make_async_copy(
                        bufs.at[gs],
                        out_hbm.at[pl.ds(base, CHUNK)],
                        wsem.at[gs],
                    ).wait()

                pltpu.async_copy(
                    table_hbm.at[idx_v.at[j + KLEAD]], bufs.at[gs], gsem.at[gs]
                )

            pltpu.make_async_copy(
                table_hbm.at[idx_v.at[j]], bufs.at[b], gsem.at[b]
            ).wait()
            pltpu.async_copy(
                bufs.at[b], out_hbm.at[pl.ds(base + j * CHUNK, CHUNK)], wsem.at[b]
            )

    for jj in range(CHUNKS - (NBUF - KLEAD), CHUNKS):
        b = jj % NBUF
        pltpu.make_async_copy(
            bufs.at[b], out_hbm.at[pl.ds(base, CHUNK)], wsem.at[b]
        ).wait()


def kernel(input, weight):
    idx = input.reshape(NW, CHUNKS, CHUNK).astype(jnp.int32)
    out = _gather_kernel(idx, weight)
    return out.reshape(BATCH, SEQ, DIM)


# speedup vs baseline: 1.0444x; 1.0003x over previous
import functools
import math

import jax
import jax.numpy as jnp
from jax import lax
from jax.experimental import pallas as pl
from jax.experimental.pallas import tpu as pltpu
from jax.experimental.pallas import tpu_sc as plsc

NUM_ROWS = 1000000
DIM = 64
BATCH = 4096
SEQ = 50

_info = plsc.get_sparse_core_info()
NC, NS = _info.num_cores, _info.num_subcores
NW = NC * NS

B = BATCH * SEQ
B_PER_W = B // NW
CHUNK = 128
CHUNKS = B_PER_W // CHUNK

_mesh = plsc.VectorSubcoreMesh(core_axis_name="c", subcore_axis_name="s")


NBUF = 10
KLEAD = 8


@functools.partial(
    pl.kernel,
    out_type=jax.ShapeDtypeStruct((B, DIM), jnp.float32),
    mesh=_mesh,
    scratch_types=[
        pltpu.VMEM((CHUNKS, CHUNK), jnp.int32),
        pltpu.VMEM((NBUF, CHUNK, DIM), jnp.float32),
        pltpu.SemaphoreType.DMA((NBUF,)),
        pltpu.SemaphoreType.DMA((NBUF,)),
    ],
    compiler_params=pltpu.CompilerParams(use_tc_tiling_on_sc=False),
)
def _gather_kernel(idx_hbm, table_hbm, out_hbm, idx_v, bufs, gsem, wsem):
    wid = lax.axis_index("s") * NC + lax.axis_index("c")
    pltpu.sync_copy(idx_hbm.at[wid], idx_v)
    base = wid * B_PER_W

    for jj in range(KLEAD):
        pltpu.async_copy(
            table_hbm.at[idx_v.at[jj]], bufs.at[jj % NBUF], gsem.at[jj % NBUF]
        )

    @pl.loop(0, CHUNKS, step=NBUF)
    def _group(g):
        for b in range(NBUF):
            j = g + b
            gs = (b + KLEAD) % NBUF

            @pl.when(j + KLEAD < CHUNKS)
            def _():
                @pl.when(j + KLEAD >= NBUF)
                def _():
                    pltpu.make_async_copy(
                        bufs.at[gs],
                        out_hbm.at[pl.ds(base, CHUNK)],
                        wsem.at[gs],
                    ).wait()

                pltpu.async_copy(
                    table_hbm.at[idx_v.at[j + KLEAD]], bufs.at[gs], gsem.at[gs]
                )

            pltpu.make_async_copy(
                table_hbm.at[idx_v.at[j]], bufs.at[b], gsem.at[b]
            ).wait()
            pltpu.async_copy(
                bufs.at[b], out_hbm.at[pl.ds(base + j * CHUNK, CHUNK)], wsem.at[b]
            )

    for jj in range(CHUNKS - (NBUF - KLEAD), CHUNKS):
        b = jj % NBUF
        pltpu.make_async_copy(
            bufs.at[b], out_hbm.at[pl.ds(base, CHUNK)], wsem.at[b]
        ).wait()


def kernel(input, weight):
    idx = input.reshape(NW, CHUNKS, CHUNK).astype(jnp.int32)
    out = _gather_kernel(idx, weight)
    return out.reshape(BATCH, SEQ, DIM)
